# baseline (speedup 1.0000x reference)
import functools

import jax
import jax.numpy as jnp
from jax.experimental import pallas as pl
from jax.experimental.pallas import tpu as pltpu
from jax.sharding import Mesh, PartitionSpec as P

try:
    _shard_map = functools.partial(jax.shard_map, check_vma=False)
except AttributeError:
    from jax.experimental.shard_map import shard_map as _shard_map

_N = 2048
_D = 128
_P = 8
_TOPK = 32
_PD = _P * _D
_BLK = 256
_GRID = _N // _BLK
_BISECT_ITERS = 21


def _prep_kernel(f_ref, w_ref, y_ref, yt_ref):
    f = f_ref[...]
    w = w_ref[...]
    cols = []
    for p in range(_P):
        fw = f * w[p:p + 1, :]
        n = jnp.sqrt(jnp.sum(fw * fw, axis=1, keepdims=True))
        cols.append(fw / jnp.maximum(n, 1e-12))
    y = jnp.concatenate(cols, axis=1)
    y_ref[...] = y
    yt_ref[...] = y.T


def _sim_kernel(y_ref, yt_ref, s_ref, t_ref):
    s = jax.lax.dot_general(
        y_ref[...], yt_ref[...], (((1,), (0,)), ((), ())),
        preferred_element_type=jnp.float32) * (1.0 / _P)
    s_ref[...] = s

    def body(_, carry):
        lo, hi = carry
        mid = (lo + hi) * 0.5
        cnt = jnp.count_nonzero(s >= mid, axis=1, keepdims=True)
        pred = cnt >= _TOPK
        return jnp.where(pred, mid, lo), jnp.where(pred, hi, mid)

    lo, _ = jax.lax.fori_loop(
        0, _BISECT_ITERS, body,
        (jnp.full((_BLK, 1), -1.25, jnp.float32),
         jnp.full((_BLK, 1), 1.25, jnp.float32)))
    t_ref[...] = lo


def _mask_kernel(s_ref, tc_ref, tr_ref, o_ref):
    s = s_ref[...]
    ti = tc_ref[...]
    tj = tr_ref[...]
    keep = (s >= ti).astype(jnp.float32) + (s >= tj).astype(jnp.float32)
    o_ref[...] = s * keep * 0.5


def _prep_call(features, weight_tensor):
    return pl.pallas_call(
        _prep_kernel,
        grid=(_GRID,),
        in_specs=[
            pl.BlockSpec((_BLK, _D), lambda i: (i, 0)),
            pl.BlockSpec((_P, _D), lambda i: (0, 0)),
        ],
        out_specs=[
            pl.BlockSpec((_BLK, _PD), lambda i: (i, 0)),
            pl.BlockSpec((_PD, _BLK), lambda i: (0, i)),
        ],
        out_shape=[
            jax.ShapeDtypeStruct((_N, _PD), jnp.float32),
            jax.ShapeDtypeStruct((_PD, _N), jnp.float32),
        ],
    )(features, weight_tensor)


def _sim_call(y_rows, yt, n_rows):
    return pl.pallas_call(
        _sim_kernel,
        grid=(n_rows // _BLK,),
        in_specs=[
            pl.BlockSpec((_BLK, _PD), lambda i: (i, 0)),
            pl.BlockSpec((_PD, _N), lambda i: (0, 0)),
        ],
        out_specs=[
            pl.BlockSpec((_BLK, _N), lambda i: (i, 0)),
            pl.BlockSpec((_BLK, 1), lambda i: (i, 0)),
        ],
        out_shape=[
            jax.ShapeDtypeStruct((n_rows, _N), jnp.float32),
            jax.ShapeDtypeStruct((n_rows, 1), jnp.float32),
        ],
    )(y_rows, yt)


def _mask_call(s, tcol, trow, n_rows):
    return pl.pallas_call(
        _mask_kernel,
        grid=(n_rows // _BLK,),
        in_specs=[
            pl.BlockSpec((_BLK, _N), lambda i: (i, 0)),
            pl.BlockSpec((_BLK, 1), lambda i: (i, 0)),
            pl.BlockSpec((1, _N), lambda i: (0, 0)),
        ],
        out_specs=pl.BlockSpec((_BLK, _N), lambda i: (i, 0)),
        out_shape=jax.ShapeDtypeStruct((n_rows, _N), jnp.float32),
    )(s, tcol, trow)


def _one_device(features, weight_tensor):
    y, yt = _prep_call(features, weight_tensor)
    s, tcol = _sim_call(y, yt, _N)
    out = _mask_call(s, tcol, tcol.reshape(1, _N), _N)
    return out


def _two_device_body(features, weight_tensor):
    half = _N // 2
    idx = jax.lax.axis_index("x")
    y, yt = _prep_call(features, weight_tensor)
    y_rows = jax.lax.dynamic_slice(y, (idx * half, 0), (half, _PD))
    s, tcol = _sim_call(y_rows, yt, half)
    t_full = jax.lax.all_gather(tcol, "x", axis=0, tiled=True)
    out = _mask_call(s, tcol, t_full.reshape(1, _N), half)
    return out


def kernel(features, weight_tensor):
    devs = jax.devices()
    if len(devs) >= 2:
        mesh = Mesh(devs[:2], ("x",))
        fn = _shard_map(
            _two_device_body, mesh=mesh,
            in_specs=(P(None, None), P(None, None)),
            out_specs=P("x", None))
        return fn(features, weight_tensor)
    return _one_device(features, weight_tensor)


# speedup vs baseline: 2.5513x; 2.5513x over previous
import jax
import jax.numpy as jnp
from jax.experimental import pallas as pl
from jax.experimental.pallas import tpu as pltpu

_N = 2048
_D = 128
_P = 8
_TOPK = 32
_PD = _P * _D
_BLK = 256
_GRID = _N // _BLK
_BISECT_ITERS = 21


def _fused_kernel(f_ref, w_ref, o_ref, y_s, yt_s, s_s, tc_s, tr_s):
    pid = pl.program_id(0)

    @pl.when(pid < _GRID)
    def _prep():
        f = f_ref[...]
        w = w_ref[...]
        cols = []
        for p in range(_P):
            fw = f * w[p:p + 1, :]
            n = jnp.sqrt(jnp.sum(fw * fw, axis=1, keepdims=True))
            cols.append(fw / jnp.maximum(n, 1e-12))
        y = jnp.concatenate(cols, axis=1)
        row = pid * _BLK
        y_s[pl.ds(row, _BLK), :] = y
        yt_s[:, pl.ds(row, _BLK)] = y.T

    @pl.when((pid >= _GRID) & (pid < 2 * _GRID))
    def _sim():
        row = (pid - _GRID) * _BLK
        y = y_s[pl.ds(row, _BLK), :]
        s = jax.lax.dot_general(
            y, yt_s[...], (((1,), (0,)), ((), ())),
            preferred_element_type=jnp.float32) * (1.0 / _P)
        s_s[pl.ds(row, _BLK), :] = s

        def body(_, carry):
            lo, hi = carry
            mid = (lo + hi) * 0.5
            cnt = jnp.count_nonzero(s >= mid, axis=1, keepdims=True)
            pred = cnt >= _TOPK
            return jnp.where(pred, mid, lo), jnp.where(pred, hi, mid)

        lo, hi = jax.lax.fori_loop(
            0, _BISECT_ITERS, body,
            (jnp.full((_BLK, 1), -1.25, jnp.float32),
             jnp.full((_BLK, 1), 1.25, jnp.float32)),
            unroll=3)
        tc_s[pl.ds(row, _BLK), :] = lo
        tr_s[:, pl.ds(row, _BLK)] = lo.T

    @pl.when(pid >= 2 * _GRID)
    def _mask():
        row = (pid - 2 * _GRID) * _BLK
        s = s_s[pl.ds(row, _BLK), :]
        ti = tc_s[pl.ds(row, _BLK), :]
        tj = tr_s[...]
        keep = (s >= ti).astype(jnp.float32) + (s >= tj).astype(jnp.float32)
        o_ref[...] = s * keep * 0.5


@jax.jit
def kernel(features, weight_tensor):
    return pl.pallas_call(
        _fused_kernel,
        grid=(3 * _GRID,),
        in_specs=[
            pl.BlockSpec((_BLK, _D), lambda i: (jnp.minimum(i, _GRID - 1), 0)),
            pl.BlockSpec((_P, _D), lambda i: (0, 0)),
        ],
        out_specs=pl.BlockSpec(
            (_BLK, _N), lambda i: (jnp.maximum(i - 2 * _GRID, 0), 0)),
        out_shape=jax.ShapeDtypeStruct((_N, _N), jnp.float32),
        scratch_shapes=[
            pltpu.VMEM((_N, _PD), jnp.float32),
            pltpu.VMEM((_PD, _N), jnp.float32),
            pltpu.VMEM((_N, _N), jnp.float32),
            pltpu.VMEM((_N, 1), jnp.float32),
            pltpu.VMEM((1, _N), jnp.float32),
        ],
    )(features, weight_tensor)


# speedup vs baseline: 2.6693x; 1.0463x over previous
import jax
import jax.numpy as jnp
from jax.experimental import pallas as pl
from jax.experimental.pallas import tpu as pltpu

_N = 2048
_D = 128
_P = 8
_TOPK = 32
_PD = _P * _D
_BLK = 256
_GRID = _N // _BLK
_BISECT_ITERS = 21


def _fused_kernel(f_ref, w_ref, o_ref, y_s, yt_s, s_s, tc_s, tr_s):
    pid = pl.program_id(0)

    @pl.when(pid < _GRID)
    def _prep():
        f = f_ref[...]
        w = w_ref[...]
        cols = []
        for p in range(_P):
            fw = f * w[p:p + 1, :]
            n = jnp.sqrt(jnp.sum(fw * fw, axis=1, keepdims=True))
            cols.append(fw / jnp.maximum(n, 1e-12))
        y = jnp.concatenate(cols, axis=1)
        row = pid * _BLK
        y_s[pl.ds(row, _BLK), :] = y
        yt_s[:, pl.ds(row, _BLK)] = y.T

    @pl.when((pid >= _GRID) & (pid < 2 * _GRID))
    def _sim():
        row = (pid - _GRID) * _BLK
        y = y_s[pl.ds(row, _BLK), :]
        s = jax.lax.dot_general(
            y, yt_s[...], (((1,), (0,)), ((), ())),
            preferred_element_type=jnp.float32) * (1.0 / _P)
        s_s[pl.ds(row, _BLK), :] = s

        def body(_, carry):
            lo, hi = carry
            mid = (lo + hi) * 0.5
            cnt = jnp.count_nonzero(s >= mid, axis=1, keepdims=True)
            pred = cnt >= _TOPK
            return jnp.where(pred, mid, lo), jnp.where(pred, hi, mid)

        lo, hi = jax.lax.fori_loop(
            0, _BISECT_ITERS, body,
            (jnp.full((_BLK, 1), -1.25, jnp.float32),
             jnp.full((_BLK, 1), 1.25, jnp.float32)),
            unroll=7)
        tc_s[pl.ds(row, _BLK), :] = lo
        tr_s[:, pl.ds(row, _BLK)] = lo.T

    @pl.when(pid >= 2 * _GRID)
    def _mask():
        row = (pid - 2 * _GRID) * _BLK
        s = s_s[pl.ds(row, _BLK), :]
        ti = tc_s[pl.ds(row, _BLK), :]
        tj = tr_s[...]
        keep = (s >= ti).astype(jnp.float32) + (s >= tj).astype(jnp.float32)
        o_ref[...] = s * keep * 0.5


@jax.jit
def kernel(features, weight_tensor):
    return pl.pallas_call(
        _fused_kernel,
        grid=(3 * _GRID,),
        in_specs=[
            pl.BlockSpec((_BLK, _D), lambda i: (jnp.minimum(i, _GRID - 1), 0)),
            pl.BlockSpec((_P, _D), lambda i: (0, 0)),
        ],
        out_specs=pl.BlockSpec(
            (_BLK, _N), lambda i: (jnp.maximum(i - 2 * _GRID, 0), 0)),
        out_shape=jax.ShapeDtypeStruct((_N, _N), jnp.float32),
        scratch_shapes=[
            pltpu.VMEM((_N, _PD), jnp.float32),
            pltpu.VMEM((_PD, _N), jnp.float32),
            pltpu.VMEM((_N, _N), jnp.float32),
            pltpu.VMEM((_N, 1), jnp.float32),
            pltpu.VMEM((1, _N), jnp.float32),
        ],
    )(features, weight_tensor)


# speedup vs baseline: 2.8212x; 1.0569x over previous
import jax
import jax.numpy as jnp
from jax.experimental import pallas as pl
from jax.experimental.pallas import tpu as pltpu

_N = 2048
_D = 128
_P = 8
_TOPK = 32
_PD = _P * _D
_BLK = 256
_GRID = _N // _BLK
_BISECT_ITERS = 21


def _fused_kernel(f_ref, w_ref, o_ref, y_s, yt_s, s_s, tc_s, tr_s):
    pid = pl.program_id(0)

    @pl.when(pid < _GRID)
    def _prep():
        f = f_ref[...]
        w = w_ref[...]
        cols = []
        for p in range(_P):
            fw = f * w[p:p + 1, :]
            n = jnp.sqrt(jnp.sum(fw * fw, axis=1, keepdims=True))
            cols.append(fw / jnp.maximum(n, 1e-12))
        y = jnp.concatenate(cols, axis=1)
        row = pid * _BLK
        y_s[pl.ds(row, _BLK), :] = y
        yt_s[:, pl.ds(row, _BLK)] = y.T

    @pl.when((pid >= _GRID) & (pid < 2 * _GRID))
    def _sim():
        row = (pid - _GRID) * _BLK
        y = y_s[pl.ds(row, _BLK), :]
        s = jax.lax.dot_general(
            y, yt_s[...], (((1,), (0,)), ((), ())),
            preferred_element_type=jnp.float32) * (1.0 / _P)
        s_s[pl.ds(row, _BLK), :] = s

        def body(_, carry):
            lo, hi = carry
            mid = (lo + hi) * 0.5
            cnt = jnp.count_nonzero(s >= mid, axis=1, keepdims=True)
            pred = cnt >= _TOPK
            return jnp.where(pred, mid, lo), jnp.where(pred, hi, mid)

        lo, hi = jax.lax.fori_loop(
            0, _BISECT_ITERS, body,
            (jnp.full((_BLK, 1), -1.25, jnp.float32),
             jnp.full((_BLK, 1), 1.25, jnp.float32)),
            unroll=21)
        tc_s[pl.ds(row, _BLK), :] = lo
        tr_s[:, pl.ds(row, _BLK)] = lo.T

    @pl.when(pid >= 2 * _GRID)
    def _mask():
        row = (pid - 2 * _GRID) * _BLK
        s = s_s[pl.ds(row, _BLK), :]
        ti = tc_s[pl.ds(row, _BLK), :]
        tj = tr_s[...]
        keep = (s >= ti).astype(jnp.float32) + (s >= tj).astype(jnp.float32)
        o_ref[...] = s * keep * 0.5


@jax.jit
def kernel(features, weight_tensor):
    return pl.pallas_call(
        _fused_kernel,
        grid=(3 * _GRID,),
        in_specs=[
            pl.BlockSpec((_BLK, _D), lambda i: (jnp.minimum(i, _GRID - 1), 0)),
            pl.BlockSpec((_P, _D), lambda i: (0, 0)),
        ],
        out_specs=pl.BlockSpec(
            (_BLK, _N), lambda i: (jnp.maximum(i - 2 * _GRID, 0), 0)),
        out_shape=jax.ShapeDtypeStruct((_N, _N), jnp.float32),
        scratch_shapes=[
            pltpu.VMEM((_N, _PD), jnp.float32),
            pltpu.VMEM((_PD, _N), jnp.float32),
            pltpu.VMEM((_N, _N), jnp.float32),
            pltpu.VMEM((_N, 1), jnp.float32),
            pltpu.VMEM((1, _N), jnp.float32),
        ],
    )(features, weight_tensor)
